# baseline (speedup 1.0000x reference)
import functools

import jax
import jax.numpy as jnp
from jax import lax
from jax.experimental import pallas as pl
from jax.experimental.pallas import tpu as pltpu

NT = 256
C = 128
UNROLL = 4


def _round_up(v: int, m: int) -> int:
    return (v + m - 1) // m * m


def _project_kernel(x_ref, deg_ref, w_ref, h_ref):
    h = jnp.dot(x_ref[...].astype(jnp.bfloat16),
                w_ref[...].astype(jnp.bfloat16),
                preferred_element_type=jnp.float32)
    h_ref[...] = (h * lax.rsqrt(deg_ref[...])).astype(h_ref.dtype)


def _aggregate_kernel(cb_ref, ck_ref, key_ref, h_ref, deg_ref, b_ref,
                      o_ref, acc_ref, *, n_tiles: int):
    i = pl.program_id(0)
    start = cb_ref[i * n_tiles]
    end = cb_ref[i * n_tiles + n_tiles]

    acc_ref[...] = h_ref[pl.ds(i * NT, NT), :].astype(jnp.float32)

    riota = lax.broadcasted_iota(jnp.int32, (NT, C), 0)

    def one_chunk(c):
        k = ck_ref[c]
        kb = key_ref[pl.ds(c, 1), :]
        dl = jnp.where(kb == 0, NT, (kb >> 8) & 511)
        sl = kb & 255
        d_t = (riota == dl).astype(jnp.bfloat16)
        s_t = (riota == sl).astype(jnp.bfloat16)
        h_k = h_ref[pl.ds(k * NT, NT), :]
        g = lax.dot_general(s_t, h_k, (((0,), (0,)), ((), ())),
                            preferred_element_type=jnp.float32)
        return d_t, g.astype(jnp.bfloat16)

    def body(gidx, carry):
        c0 = start + gidx * UNROLL
        parts = [one_chunk(c0 + u) for u in range(UNROLL)]
        d_cat = jnp.concatenate([p[0] for p in parts], axis=1)
        g_cat = jnp.concatenate([p[1] for p in parts], axis=0)
        acc_ref[...] += jnp.dot(d_cat, g_cat,
                                preferred_element_type=jnp.float32)
        return carry

    lax.fori_loop(0, (end - start) // UNROLL, body, 0)
    o_ref[...] = acc_ref[...] * lax.rsqrt(deg_ref[...]) + b_ref[...]


def kernel(x, edge_index, weight, bias):
    N, D_in = x.shape
    D_out = weight.shape[1]
    E = edge_index.shape[1]

    n_p = _round_up(max(N, NT), NT)
    n_tiles = n_p // NT
    n_buckets = n_tiles * n_tiles
    d_in_p = _round_up(D_in, 128)
    d_out_p = _round_up(D_out, 128)

    src = edge_index[0].astype(jnp.int32)
    dst = edge_index[1].astype(jnp.int32)
    dst0 = dst

    E_pad = _round_up(max(E, C), C)
    pad = E_pad - E
    bucket = (dst // NT) * n_tiles + (src // NT)
    if pad:
        src = jnp.concatenate([src, jnp.zeros((pad,), jnp.int32)])
        dst = jnp.concatenate([dst, jnp.zeros((pad,), jnp.int32)])
        bucket = jnp.concatenate(
            [bucket, jnp.full((pad,), n_buckets, jnp.int32)])

    tcm = _round_up(n_buckets + E_pad // C + 1 + n_tiles * (UNROLL - 1), 8)
    key = (1 << 30) | (bucket << 17) | ((dst % NT) << 8) | (src % NT)

    n_ec = E_pad // C
    b2 = bucket.reshape(n_ec, C)
    nh = n_ec * (n_buckets + 1)
    flat_idx = jnp.concatenate([
        (jnp.arange(n_ec, dtype=jnp.int32)[:, None] * (n_buckets + 1)
         + b2).ravel(),
        nh + dst0])
    combo = jnp.zeros((nh + n_p,), jnp.int32).at[flat_idx].add(1)
    hist = combo[:nh].reshape(n_ec, n_buckets + 1)
    deg_p = (combo[nh:].astype(jnp.float32) + 1.0)[:, None]
    counts = jnp.sum(hist, axis=0)
    nch = (counts + C - 1) // C
    nch_rows = nch[:n_buckets].reshape(n_tiles, n_tiles)
    row_pad = (-jnp.sum(nch_rows, axis=1)) % UNROLL
    nch_rows = nch_rows.at[:, -1].add(row_pad)
    nch = jnp.concatenate([nch_rows.reshape(-1), nch[n_buckets:]])
    chunk_base = jnp.concatenate(
        [jnp.zeros((1,), jnp.int32), jnp.cumsum(nch, dtype=jnp.int32)])
    ck = (jnp.repeat(jnp.arange(n_buckets + 1, dtype=jnp.int32), nch,
                     total_repeat_length=tcm) % n_tiles).astype(jnp.int32)

    ar = jnp.arange(n_ec)
    tril = (ar[:, None] > ar[None, :]).astype(jnp.float32)
    prefix = jax.lax.dot(tril, hist.astype(jnp.float32),
                         precision=jax.lax.Precision.HIGHEST
                         ).astype(jnp.int32)
    prefix = prefix + chunk_base[None, :n_buckets + 1] * C
    eq = b2[:, :, None] == b2[:, None, :]
    tri = jnp.arange(C)[None, :] < jnp.arange(C)[:, None]
    within = jnp.sum(eq & tri[None], axis=2, dtype=jnp.int32)
    pos = prefix[jnp.arange(n_ec)[:, None], b2] + within
    key_pad = jnp.zeros((tcm * C,), jnp.int32).at[pos.ravel()].add(
        key).reshape(tcm, C)

    if (N, D_in) == (n_p, d_in_p):
        x_p = x
    else:
        x_p = jnp.zeros((n_p, d_in_p), x.dtype).at[:N, :D_in].set(x)
    if (D_in, D_out) == (d_in_p, d_out_p):
        w_p = weight
    else:
        w_p = jnp.zeros((d_in_p, d_out_p), weight.dtype).at[
            :D_in, :D_out].set(weight)
    if D_out == d_out_p:
        b_p = bias.astype(jnp.float32)[None, :]
    else:
        b_p = jnp.zeros((1, d_out_p), jnp.float32).at[0, :D_out].set(
            bias.astype(jnp.float32))

    h_scaled = pl.pallas_call(
        _project_kernel,
        out_shape=jax.ShapeDtypeStruct((n_p, d_out_p), jnp.bfloat16),
        grid_spec=pltpu.PrefetchScalarGridSpec(
            num_scalar_prefetch=0,
            grid=(n_tiles,),
            in_specs=[
                pl.BlockSpec((NT, d_in_p), lambda i: (i, 0)),
                pl.BlockSpec((NT, 1), lambda i: (i, 0)),
                pl.BlockSpec((d_in_p, d_out_p), lambda i: (0, 0)),
            ],
            out_specs=pl.BlockSpec((NT, d_out_p), lambda i: (i, 0)),
        ),
        compiler_params=pltpu.CompilerParams(
            dimension_semantics=("parallel",),
        ),
    )(x_p, deg_p, w_p)

    out_p = pl.pallas_call(
        functools.partial(_aggregate_kernel, n_tiles=n_tiles),
        out_shape=jax.ShapeDtypeStruct((n_p, d_out_p), jnp.float32),
        grid_spec=pltpu.PrefetchScalarGridSpec(
            num_scalar_prefetch=2,
            grid=(n_tiles,),
            in_specs=[
                pl.BlockSpec((tcm, C), lambda i, *_: (0, 0)),
                pl.BlockSpec((n_p, d_out_p), lambda i, *_: (0, 0)),
                pl.BlockSpec((NT, 1), lambda i, *_: (i, 0)),
                pl.BlockSpec((1, d_out_p), lambda i, *_: (0, 0)),
            ],
            out_specs=pl.BlockSpec((NT, d_out_p), lambda i, *_: (i, 0)),
            scratch_shapes=[pltpu.VMEM((NT, d_out_p), jnp.float32)],
        ),
        compiler_params=pltpu.CompilerParams(
            dimension_semantics=("parallel",),
        ),
    )(chunk_base, ck, key_pad, h_scaled, deg_p, b_p)

    return out_p[:N, :D_out]


# speedup vs baseline: 1.1788x; 1.1788x over previous
import functools

import jax
import jax.numpy as jnp
from jax import lax
from jax.experimental import pallas as pl
from jax.experimental.pallas import tpu as pltpu

NT = 256
C = 128
UNROLL = 8


def _round_up(v: int, m: int) -> int:
    return (v + m - 1) // m * m


def _project_kernel(x_ref, deg_ref, w_ref, h_ref):
    h = jnp.dot(x_ref[...].astype(jnp.bfloat16),
                w_ref[...].astype(jnp.bfloat16),
                preferred_element_type=jnp.float32)
    h_ref[...] = (h * lax.rsqrt(deg_ref[...])).astype(h_ref.dtype)


def _aggregate_kernel(cb_ref, ck_ref, key_ref, h_ref, deg_ref, b_ref,
                      o_ref, acc_ref, *, n_tiles: int):
    i = pl.program_id(0)
    start = cb_ref[i * n_tiles]
    end = cb_ref[i * n_tiles + n_tiles]

    acc_ref[...] = h_ref[pl.ds(i * NT, NT), :].astype(jnp.float32)

    riota = lax.broadcasted_iota(jnp.int32, (NT, C), 0)

    def one_chunk(c):
        k = ck_ref[c]
        kb = key_ref[pl.ds(c, 1), :]
        dl = jnp.where(kb == 0, NT, (kb >> 8) & 511)
        sl = kb & 255
        d_t = (riota == dl).astype(jnp.bfloat16)
        s_t = (riota == sl).astype(jnp.bfloat16)
        h_k = h_ref[pl.ds(k * NT, NT), :]
        g = lax.dot_general(s_t, h_k, (((0,), (0,)), ((), ())),
                            preferred_element_type=jnp.float32)
        return d_t, g.astype(jnp.bfloat16)

    def body(gidx, carry):
        c0 = start + gidx * UNROLL
        parts = [one_chunk(c0 + u) for u in range(UNROLL)]
        d_cat = jnp.concatenate([p[0] for p in parts], axis=1)
        g_cat = jnp.concatenate([p[1] for p in parts], axis=0)
        acc_ref[...] += jnp.dot(d_cat, g_cat,
                                preferred_element_type=jnp.float32)
        return carry

    lax.fori_loop(0, (end - start) // UNROLL, body, 0)
    o_ref[...] = acc_ref[...] * lax.rsqrt(deg_ref[...]) + b_ref[...]


def kernel(x, edge_index, weight, bias):
    N, D_in = x.shape
    D_out = weight.shape[1]
    E = edge_index.shape[1]

    n_p = _round_up(max(N, NT), NT)
    n_tiles = n_p // NT
    n_buckets = n_tiles * n_tiles
    d_in_p = _round_up(D_in, 128)
    d_out_p = _round_up(D_out, 128)

    src = edge_index[0].astype(jnp.int32)
    dst = edge_index[1].astype(jnp.int32)
    dst0 = dst

    E_pad = _round_up(max(E, C), C)
    pad = E_pad - E
    bucket = (dst // NT) * n_tiles + (src // NT)
    if pad:
        src = jnp.concatenate([src, jnp.zeros((pad,), jnp.int32)])
        dst = jnp.concatenate([dst, jnp.zeros((pad,), jnp.int32)])
        bucket = jnp.concatenate(
            [bucket, jnp.full((pad,), n_buckets, jnp.int32)])

    tcm = _round_up(n_buckets + E_pad // C + 1 + n_tiles * (UNROLL - 1), 8)
    key = (1 << 30) | (bucket << 17) | ((dst % NT) << 8) | (src % NT)

    n_ec = E_pad // C
    b2 = bucket.reshape(n_ec, C)
    nh = n_ec * (n_buckets + 1)
    flat_idx = jnp.concatenate([
        (jnp.arange(n_ec, dtype=jnp.int32)[:, None] * (n_buckets + 1)
         + b2).ravel(),
        nh + dst0])
    combo = jnp.zeros((nh + n_p,), jnp.int32).at[flat_idx].add(1)
    hist = combo[:nh].reshape(n_ec, n_buckets + 1)
    deg_p = (combo[nh:].astype(jnp.float32) + 1.0)[:, None]
    counts = jnp.sum(hist, axis=0)
    nch = (counts + C - 1) // C
    nch_rows = nch[:n_buckets].reshape(n_tiles, n_tiles)
    row_pad = (-jnp.sum(nch_rows, axis=1)) % UNROLL
    nch_rows = nch_rows.at[:, -1].add(row_pad)
    nch = jnp.concatenate([nch_rows.reshape(-1), nch[n_buckets:]])
    chunk_base = jnp.concatenate(
        [jnp.zeros((1,), jnp.int32), jnp.cumsum(nch, dtype=jnp.int32)])
    ck = (jnp.repeat(jnp.arange(n_buckets + 1, dtype=jnp.int32), nch,
                     total_repeat_length=tcm) % n_tiles).astype(jnp.int32)

    ar = jnp.arange(n_ec)
    tril = (ar[:, None] > ar[None, :]).astype(jnp.float32)
    prefix = jax.lax.dot(tril, hist.astype(jnp.float32),
                         precision=jax.lax.Precision.HIGHEST
                         ).astype(jnp.int32)
    prefix = prefix + chunk_base[None, :n_buckets + 1] * C
    eq = b2[:, :, None] == b2[:, None, :]
    tri = jnp.arange(C)[None, :] < jnp.arange(C)[:, None]
    within = jnp.sum(eq & tri[None], axis=2, dtype=jnp.int32)
    pos = prefix[jnp.arange(n_ec)[:, None], b2] + within
    key_pad = jnp.zeros((tcm * C,), jnp.int32).at[pos.ravel()].add(
        key).reshape(tcm, C)

    if (N, D_in) == (n_p, d_in_p):
        x_p = x
    else:
        x_p = jnp.zeros((n_p, d_in_p), x.dtype).at[:N, :D_in].set(x)
    if (D_in, D_out) == (d_in_p, d_out_p):
        w_p = weight
    else:
        w_p = jnp.zeros((d_in_p, d_out_p), weight.dtype).at[
            :D_in, :D_out].set(weight)
    if D_out == d_out_p:
        b_p = bias.astype(jnp.float32)[None, :]
    else:
        b_p = jnp.zeros((1, d_out_p), jnp.float32).at[0, :D_out].set(
            bias.astype(jnp.float32))

    h_scaled = pl.pallas_call(
        _project_kernel,
        out_shape=jax.ShapeDtypeStruct((n_p, d_out_p), jnp.bfloat16),
        grid_spec=pltpu.PrefetchScalarGridSpec(
            num_scalar_prefetch=0,
            grid=(n_tiles,),
            in_specs=[
                pl.BlockSpec((NT, d_in_p), lambda i: (i, 0)),
                pl.BlockSpec((NT, 1), lambda i: (i, 0)),
                pl.BlockSpec((d_in_p, d_out_p), lambda i: (0, 0)),
            ],
            out_specs=pl.BlockSpec((NT, d_out_p), lambda i: (i, 0)),
        ),
        compiler_params=pltpu.CompilerParams(
            dimension_semantics=("parallel",),
        ),
    )(x_p, deg_p, w_p)

    out_p = pl.pallas_call(
        functools.partial(_aggregate_kernel, n_tiles=n_tiles),
        out_shape=jax.ShapeDtypeStruct((n_p, d_out_p), jnp.float32),
        grid_spec=pltpu.PrefetchScalarGridSpec(
            num_scalar_prefetch=2,
            grid=(n_tiles,),
            in_specs=[
                pl.BlockSpec((tcm, C), lambda i, *_: (0, 0)),
                pl.BlockSpec((n_p, d_out_p), lambda i, *_: (0, 0)),
                pl.BlockSpec((NT, 1), lambda i, *_: (i, 0)),
                pl.BlockSpec((1, d_out_p), lambda i, *_: (0, 0)),
            ],
            out_specs=pl.BlockSpec((NT, d_out_p), lambda i, *_: (i, 0)),
            scratch_shapes=[pltpu.VMEM((NT, d_out_p), jnp.float32)],
        ),
        compiler_params=pltpu.CompilerParams(
            dimension_semantics=("parallel",),
        ),
    )(chunk_base, ck, key_pad, h_scaled, deg_p, b_p)

    return out_p[:N, :D_out]
